# baseline (speedup 1.0000x reference)
import functools

import jax
import jax.numpy as jnp
from jax import lax
from jax.experimental import pallas as pl
from jax.experimental.pallas import tpu as pltpu
from jax.experimental.pallas import tpu_sc as plsc

VOCAB = 1000000
EMBED_DIM = 32
BATCH = 16384
HIST = 50
DROPOUT = 0.1

NC = 2
NS = 16
NW = NC * NS
L = 16

C = BATCH // NW
J = C // 128


def _make_stages():
    lane = lax.iota(jnp.int32, L)
    stages = []
    for s in (1, 2, 4, 8):
        stages.append((
            s,
            (lane & s) == 0,
            (lane - s) & (L - 1),
            (lane + s) & (L - 1),
        ))
    return stages


def _transpose16(v, stages):
    for s, keep, idxm, idxp in stages:
        nv = list(v)
        for i in range(L):
            if i & s:
                continue
            a, b = v[i], v[i | s]
            bg = b.at[idxm].get(mode="promise_in_bounds")
            ag = a.at[idxp].get(mode="promise_in_bounds")
            nv[i] = jnp.where(keep, a, bg)
            nv[i | s] = jnp.where(keep, ag, b)
        v = nv
    return v


def _body(words_ref, weight_ref, out_ref,
          idx_b, rows_b, out_b, sem_g, sem_o):
    wid = lax.axis_index("s") * NC + lax.axis_index("c")
    b0 = wid * C
    stages = _make_stages()

    def fire_gathers(h, par):
        pltpu.sync_copy(words_ref.at[h, pl.ds(b0, C)], idx_b[par])
        for j in range(J):
            sl = pl.ds(j * 128, 128)
            pltpu.async_copy(weight_ref.at[idx_b[par].at[sl]],
                             rows_b[par].at[sl], sem_g[par])

    def wait_gathers(par):
        for j in range(J):
            sl = pl.ds(j * 128, 128)
            pltpu.make_async_copy(weight_ref.at[pl.ds(0, 128)],
                                  rows_b[par].at[sl], sem_g[par]).wait()

    def compute(h, par):
        def kb_body(kb, c2):
            k0 = kb * L
            for hh in range(EMBED_DIM // L):
                v = [rows_b[par][k0 + i, pl.ds(hh * L, L)] for i in range(L)]
                t = _transpose16(v, stages)
                for jj in range(L):
                    out_b[par][hh * L + jj, pl.ds(k0, L)] = t[jj]
            return c2

        lax.fori_loop(0, C // L, kb_body, 0)

    fire_gathers(0, 0)

    def g_body(g, carry):
        for par in range(2):
            h = 2 * g + par
            nxt = 1 - par
            if par == 0:
                fire_gathers(h + 1, nxt)
            else:
                @pl.when(g < HIST // 2 - 1)
                def _():
                    fire_gathers(h + 1, nxt)
            wait_gathers(par)

            @pl.when(g >= 1)
            def _():
                pltpu.make_async_copy(
                    out_b[par], out_ref.at[0, :, pl.ds(b0, C)], sem_o[par]
                ).wait()

            compute(h, par)
            pltpu.async_copy(out_b[par], out_ref.at[h, :, pl.ds(b0, C)],
                             sem_o[par])
        return carry

    lax.fori_loop(0, HIST // 2, g_body, 0)
    for par in range(2):
        pltpu.make_async_copy(out_b[par], out_ref.at[0, :, pl.ds(b0, C)],
                              sem_o[par]).wait()


@functools.partial(
    pl.kernel,
    out_type=jax.ShapeDtypeStruct((HIST, EMBED_DIM, BATCH), jnp.float32),
    mesh=plsc.VectorSubcoreMesh(core_axis_name="c", subcore_axis_name="s"),
    scratch_types=[
        pltpu.VMEM((C,), jnp.int32),
        pltpu.VMEM((C,), jnp.int32),
        pltpu.VMEM((C, EMBED_DIM), jnp.float32),
        pltpu.VMEM((C, EMBED_DIM), jnp.float32),
        pltpu.VMEM((EMBED_DIM, C), jnp.float32),
        pltpu.VMEM((EMBED_DIM, C), jnp.float32),
        pltpu.SemaphoreType.DMA,
        pltpu.SemaphoreType.DMA,
        pltpu.SemaphoreType.DMA,
        pltpu.SemaphoreType.DMA,
    ],
    compiler_params=pltpu.CompilerParams(use_tc_tiling_on_sc=False),
)
def _sc_lookup(words_ref, weight_ref, out_ref,
               idx0, idx1, rows0, rows1, outv0, outv1,
               sg0, sg1, so0, so1):
    _body(words_ref, weight_ref, out_ref,
          (idx0, idx1), (rows0, rows1), (outv0, outv1),
          (sg0, sg1), (so0, so1))


SB = 128
NSB_FULL = VOCAB // SB
NSB_MAIN = (NSB_FULL // NW) & ~1


def _tr_body(wt_ref, scale_ref, tail_ref, out_ref, in_b, sc_b, out_b, sem_i, sem_o):
    wid = lax.axis_index("s") * NC + lax.axis_index("c")
    stages = _make_stages()

    def sb_col0(i):
        return (wid + NW * i) * SB

    def fire_in(i, par):
        pltpu.async_copy(wt_ref.at[:, pl.ds(sb_col0(i), SB)], in_b[par],
                         sem_i[par])
        pltpu.async_copy(scale_ref.at[pl.ds(sb_col0(i), SB)], sc_b[par],
                         sem_i[par])

    def wait_in(par):
        pltpu.make_async_copy(wt_ref.at[:, pl.ds(0, SB)], in_b[par],
                              sem_i[par]).wait()
        pltpu.make_async_copy(scale_ref.at[pl.ds(0, SB)], sc_b[par],
                              sem_i[par]).wait()

    def compute_sb(par):
        def sub_body(sub, c2):
            m16 = sc_b[par][pl.ds(sub * L, L)]
            for hh in range(EMBED_DIM // L):
                v = [in_b[par][hh * L + i, pl.ds(sub * L, L)] * m16
                     for i in range(L)]
                t = _transpose16(v, stages)
                for jj in range(L):
                    out_b[par][pl.ds((sub * L + jj) * EMBED_DIM + hh * L, L)] \
                        = t[jj]
            return c2

        lax.fori_loop(0, SB // L, sub_body, 0)

    fire_in(0, 0)

    def g_body(g, carry):
        for par in range(2):
            i = 2 * g + par
            nxt = 1 - par
            if par == 0:
                fire_in(i + 1, nxt)
            else:
                @pl.when(g < NSB_MAIN // 2 - 1)
                def _():
                    fire_in(i + 1, nxt)
            wait_in(par)

            @pl.when(g >= 1)
            def _():
                pltpu.make_async_copy(out_b[par],
                                      out_ref.at[pl.ds(0, SB * EMBED_DIM)],
                                      sem_o[par]).wait()

            compute_sb(par)
            pltpu.async_copy(out_b[par],
                             out_ref.at[pl.ds(sb_col0(i) * EMBED_DIM,
                                              SB * EMBED_DIM)],
                             sem_o[par])
        return carry

    lax.fori_loop(0, NSB_MAIN // 2, g_body, 0)
    for par in range(2):
        pltpu.make_async_copy(out_b[par], out_ref.at[pl.ds(0, SB * EMBED_DIM)],
                              sem_o[par]).wait()

    @pl.when(wid < NSB_FULL - NW * NSB_MAIN)
    def _():
        pltpu.sync_copy(wt_ref.at[:, pl.ds(sb_col0(NSB_MAIN), SB)], in_b[0])
        pltpu.sync_copy(scale_ref.at[pl.ds(sb_col0(NSB_MAIN), SB)], sc_b[0])
        compute_sb(0)
        pltpu.sync_copy(out_b[0],
                        out_ref.at[pl.ds(sb_col0(NSB_MAIN) * EMBED_DIM,
                                         SB * EMBED_DIM)])

    @pl.when(wid == 0)
    def _():
        pltpu.sync_copy(tail_ref,
                        out_ref.at[pl.ds(NSB_FULL * SB * EMBED_DIM,
                                         (VOCAB - NSB_FULL * SB) * EMBED_DIM)])


@functools.partial(
    pl.kernel,
    out_type=jax.ShapeDtypeStruct((VOCAB * EMBED_DIM,), jnp.float32),
    mesh=plsc.VectorSubcoreMesh(core_axis_name="c", subcore_axis_name="s"),
    scratch_types=[
        pltpu.VMEM((EMBED_DIM, SB), jnp.float32),
        pltpu.VMEM((EMBED_DIM, SB), jnp.float32),
        pltpu.VMEM((SB,), jnp.float32),
        pltpu.VMEM((SB,), jnp.float32),
        pltpu.VMEM((SB * EMBED_DIM,), jnp.float32),
        pltpu.VMEM((SB * EMBED_DIM,), jnp.float32),
        pltpu.SemaphoreType.DMA,
        pltpu.SemaphoreType.DMA,
        pltpu.SemaphoreType.DMA,
        pltpu.SemaphoreType.DMA,
    ],
    compiler_params=pltpu.CompilerParams(use_tc_tiling_on_sc=True),
)
def _sc_table_rowmajor(wt_ref, scale_ref, tail_ref, out_ref, in0, in1, sc0,
                       sc1, ob0, ob1, si0, si1, so0, so1):
    _tr_body(wt_ref, scale_ref, tail_ref, out_ref, (in0, in1), (sc0, sc1),
             (ob0, ob1), (si0, si1), (so0, so1))


def kernel(words, weight):
    mask_key = jax.random.fold_in(jax.random.key(0), 12345)
    keep = jax.random.bernoulli(mask_key, p=1.0 - DROPOUT, shape=(VOCAB, 1))
    scale = keep[:, 0].astype(jnp.float32) * (1.0 / (1.0 - DROPOUT))
    words_t = words.T.astype(jnp.int32)
    w_tail = (weight[NSB_FULL * SB:, :]
              * scale[NSB_FULL * SB:, None]).reshape(-1)
    w_lin = _sc_table_rowmajor(weight.T, scale, w_tail)
    w_rm = w_lin.reshape(VOCAB, EMBED_DIM)
    out_t = _sc_lookup(words_t, w_rm)
    return jnp.transpose(out_t, (2, 0, 1))


# speedup vs baseline: 1.0296x; 1.0296x over previous
import functools

import jax
import jax.numpy as jnp
from jax import lax
from jax.experimental import pallas as pl
from jax.experimental.pallas import tpu as pltpu
from jax.experimental.pallas import tpu_sc as plsc

VOCAB = 1000000
EMBED_DIM = 32
BATCH = 16384
HIST = 50
DROPOUT = 0.1

NC = 2
NS = 16
NW = NC * NS
L = 16

C = BATCH // NW
J = C // 128


def _make_stages():
    lane = lax.iota(jnp.int32, L)
    stages = []
    for s in (1, 2, 4, 8):
        stages.append((
            s,
            (lane & s) == 0,
            (lane - s) & (L - 1),
            (lane + s) & (L - 1),
        ))
    return stages


def _transpose16(v, stages):
    for s, keep, idxm, idxp in stages:
        nv = list(v)
        for i in range(L):
            if i & s:
                continue
            a, b = v[i], v[i | s]
            bg = b.at[idxm].get(mode="promise_in_bounds")
            ag = a.at[idxp].get(mode="promise_in_bounds")
            nv[i] = jnp.where(keep, a, bg)
            nv[i | s] = jnp.where(keep, ag, b)
        v = nv
    return v


def _body(words_ref, weight_ref, scale_ref, out_ref,
          idx_b, rows_b, m_b, out_b, sem_g, sem_o):
    wid = lax.axis_index("s") * NC + lax.axis_index("c")
    b0 = wid * C
    stages = _make_stages()

    def fire_gathers(h, par):
        pltpu.sync_copy(words_ref.at[h, pl.ds(b0, C)], idx_b[par])
        for j in range(J):
            sl = pl.ds(j * 128, 128)
            pltpu.async_copy(weight_ref.at[idx_b[par].at[sl]],
                             rows_b[par].at[sl], sem_g[par])
            pltpu.async_copy(scale_ref.at[idx_b[par].at[sl]],
                             m_b[par].at[sl], sem_g[par])

    def wait_gathers(par):
        for j in range(J):
            sl = pl.ds(j * 128, 128)
            pltpu.make_async_copy(weight_ref.at[pl.ds(0, 128)],
                                  rows_b[par].at[sl], sem_g[par]).wait()
            pltpu.make_async_copy(scale_ref.at[pl.ds(0, 128)],
                                  m_b[par].at[sl], sem_g[par]).wait()

    def compute(h, par):
        def kb_body(kb, c2):
            for u in range(2):
                k0 = (2 * kb + u) * L
                m16 = m_b[par][pl.ds(k0, L)]
                for hh in range(EMBED_DIM // L):
                    v = [rows_b[par][k0 + i, pl.ds(hh * L, L)]
                         for i in range(L)]
                    t = _transpose16(v, stages)
                    for jj in range(L):
                        out_b[par][hh * L + jj, pl.ds(k0, L)] = t[jj] * m16
            return c2

        lax.fori_loop(0, C // L // 2, kb_body, 0)

    fire_gathers(0, 0)

    def g_body(g, carry):
        for par in range(2):
            h = 2 * g + par
            nxt = 1 - par
            if par == 0:
                fire_gathers(h + 1, nxt)
            else:
                @pl.when(g < HIST // 2 - 1)
                def _():
                    fire_gathers(h + 1, nxt)
            wait_gathers(par)

            @pl.when(g >= 1)
            def _():
                pltpu.make_async_copy(
                    out_b[par], out_ref.at[0, :, pl.ds(b0, C)], sem_o[par]
                ).wait()

            compute(h, par)
            pltpu.async_copy(out_b[par], out_ref.at[h, :, pl.ds(b0, C)],
                             sem_o[par])
        return carry

    lax.fori_loop(0, HIST // 2, g_body, 0)
    for par in range(2):
        pltpu.make_async_copy(out_b[par], out_ref.at[0, :, pl.ds(b0, C)],
                              sem_o[par]).wait()


@functools.partial(
    pl.kernel,
    out_type=jax.ShapeDtypeStruct((HIST, EMBED_DIM, BATCH), jnp.float32),
    mesh=plsc.VectorSubcoreMesh(core_axis_name="c", subcore_axis_name="s"),
    scratch_types=[
        pltpu.VMEM((C,), jnp.int32),
        pltpu.VMEM((C,), jnp.int32),
        pltpu.VMEM((C, EMBED_DIM), jnp.float32),
        pltpu.VMEM((C, EMBED_DIM), jnp.float32),
        pltpu.VMEM((C,), jnp.float32),
        pltpu.VMEM((C,), jnp.float32),
        pltpu.VMEM((EMBED_DIM, C), jnp.float32),
        pltpu.VMEM((EMBED_DIM, C), jnp.float32),
        pltpu.SemaphoreType.DMA,
        pltpu.SemaphoreType.DMA,
        pltpu.SemaphoreType.DMA,
        pltpu.SemaphoreType.DMA,
    ],
    compiler_params=pltpu.CompilerParams(use_tc_tiling_on_sc=False),
)
def _sc_lookup(words_ref, weight_ref, scale_ref, out_ref,
               idx0, idx1, rows0, rows1, m0, m1, outv0, outv1,
               sg0, sg1, so0, so1):
    _body(words_ref, weight_ref, scale_ref, out_ref,
          (idx0, idx1), (rows0, rows1), (m0, m1), (outv0, outv1),
          (sg0, sg1), (so0, so1))


SB = 128
NSB_FULL = VOCAB // SB
NSB_MAIN = (NSB_FULL // NW) & ~1


def _tr_body(wt_ref, tail_ref, out_ref, in_b, out_b, sem_i, sem_o):
    wid = lax.axis_index("s") * NC + lax.axis_index("c")
    stages = _make_stages()

    def sb_col0(i):
        return (wid + NW * i) * SB

    def fire_in(i, par):
        pltpu.async_copy(wt_ref.at[:, pl.ds(sb_col0(i), SB)], in_b[par],
                         sem_i[par])

    def wait_in(par):
        pltpu.make_async_copy(wt_ref.at[:, pl.ds(0, SB)], in_b[par],
                              sem_i[par]).wait()

    def compute_sb(par):
        def sub_body(sub, c2):
            for hh in range(EMBED_DIM // L):
                v = [in_b[par][hh * L + i, pl.ds(sub * L, L)]
                     for i in range(L)]
                t = _transpose16(v, stages)
                for jj in range(L):
                    out_b[par][pl.ds((sub * L + jj) * EMBED_DIM + hh * L, L)] \
                        = t[jj]
            return c2

        lax.fori_loop(0, SB // L, sub_body, 0)

    fire_in(0, 0)

    def g_body(g, carry):
        for par in range(2):
            i = 2 * g + par
            nxt = 1 - par
            if par == 0:
                fire_in(i + 1, nxt)
            else:
                @pl.when(g < NSB_MAIN // 2 - 1)
                def _():
                    fire_in(i + 1, nxt)
            wait_in(par)

            @pl.when(g >= 1)
            def _():
                pltpu.make_async_copy(out_b[par],
                                      out_ref.at[pl.ds(0, SB * EMBED_DIM)],
                                      sem_o[par]).wait()

            compute_sb(par)
            pltpu.async_copy(out_b[par],
                             out_ref.at[pl.ds(sb_col0(i) * EMBED_DIM,
                                              SB * EMBED_DIM)],
                             sem_o[par])
        return carry

    lax.fori_loop(0, NSB_MAIN // 2, g_body, 0)
    for par in range(2):
        pltpu.make_async_copy(out_b[par], out_ref.at[pl.ds(0, SB * EMBED_DIM)],
                              sem_o[par]).wait()

    @pl.when(wid < NSB_FULL - NW * NSB_MAIN)
    def _():
        pltpu.sync_copy(wt_ref.at[:, pl.ds(sb_col0(NSB_MAIN), SB)], in_b[0])
        compute_sb(0)
        pltpu.sync_copy(out_b[0],
                        out_ref.at[pl.ds(sb_col0(NSB_MAIN) * EMBED_DIM,
                                         SB * EMBED_DIM)])

    @pl.when(wid == 0)
    def _():
        pltpu.sync_copy(tail_ref,
                        out_ref.at[pl.ds(NSB_FULL * SB * EMBED_DIM,
                                         (VOCAB - NSB_FULL * SB) * EMBED_DIM)])


@functools.partial(
    pl.kernel,
    out_type=jax.ShapeDtypeStruct((VOCAB * EMBED_DIM,), jnp.float32),
    mesh=plsc.VectorSubcoreMesh(core_axis_name="c", subcore_axis_name="s"),
    scratch_types=[
        pltpu.VMEM((EMBED_DIM, SB), jnp.float32),
        pltpu.VMEM((EMBED_DIM, SB), jnp.float32),
        pltpu.VMEM((SB * EMBED_DIM,), jnp.float32),
        pltpu.VMEM((SB * EMBED_DIM,), jnp.float32),
        pltpu.SemaphoreType.DMA,
        pltpu.SemaphoreType.DMA,
        pltpu.SemaphoreType.DMA,
        pltpu.SemaphoreType.DMA,
    ],
    compiler_params=pltpu.CompilerParams(use_tc_tiling_on_sc=True),
)
def _sc_table_rowmajor(wt_ref, tail_ref, out_ref, in0, in1, ob0, ob1, si0,
                       si1, so0, so1):
    _tr_body(wt_ref, tail_ref, out_ref, (in0, in1), (ob0, ob1), (si0, si1),
             (so0, so1))


def kernel(words, weight):
    mask_key = jax.random.fold_in(jax.random.key(0), 12345)
    keep = jax.random.bernoulli(mask_key, p=1.0 - DROPOUT, shape=(VOCAB, 1))
    scale = keep[:, 0].astype(jnp.float32) * (1.0 / (1.0 - DROPOUT))
    words_t = words.T.astype(jnp.int32)
    w_tail = weight[NSB_FULL * SB:, :].reshape(-1)
    w_lin = _sc_table_rowmajor(weight.T, w_tail)
    w_rm = w_lin.reshape(VOCAB, EMBED_DIM)
    out_t = _sc_lookup(words_t, w_rm, scale)
    return jnp.transpose(out_t, (2, 0, 1))


# speedup vs baseline: 1.1236x; 1.0913x over previous
import functools

import jax
import jax.numpy as jnp
from jax import lax
from jax.experimental import pallas as pl
from jax.experimental.pallas import tpu as pltpu
from jax.experimental.pallas import tpu_sc as plsc

VOCAB = 1000000
EMBED_DIM = 32
BATCH = 16384
HIST = 50
DROPOUT = 0.1

NC = 2
NS = 16
NW = NC * NS
L = 16

C = BATCH // NW
J = C // 128


def _make_stages():
    lane = lax.iota(jnp.int32, L)
    stages = []
    for s in (1, 2, 4, 8):
        stages.append((
            s,
            (lane & s) == 0,
            (lane - s) & (L - 1),
            (lane + s) & (L - 1),
        ))
    return stages


def _transpose16(v, stages):
    for s, keep, idxm, idxp in stages:
        nv = list(v)
        for i in range(L):
            if i & s:
                continue
            a, b = v[i], v[i | s]
            bg = b.at[idxm].get(mode="promise_in_bounds")
            ag = a.at[idxp].get(mode="promise_in_bounds")
            nv[i] = jnp.where(keep, a, bg)
            nv[i | s] = jnp.where(keep, ag, b)
        v = nv
    return v


def _body(words_ref, weight_ref, scale_ref, out_ref,
          idx_b, rows_b, m_b, out_b, sem_g, sem_o):
    wid = lax.axis_index("s") * NC + lax.axis_index("c")
    b0 = wid * C
    stages = _make_stages()

    def fire_gathers(h, par):
        pltpu.sync_copy(words_ref.at[h, pl.ds(b0, C)], idx_b[par])
        for j in range(J):
            sl = pl.ds(j * 128, 128)
            pltpu.async_copy(weight_ref.at[idx_b[par].at[sl]],
                             rows_b[par].at[sl], sem_g[par])
            pltpu.async_copy(scale_ref.at[idx_b[par].at[sl]],
                             m_b[par].at[sl], sem_g[par])

    def wait_gathers(par):
        for j in range(J):
            sl = pl.ds(j * 128, 128)
            pltpu.make_async_copy(weight_ref.at[pl.ds(0, 128)],
                                  rows_b[par].at[sl], sem_g[par]).wait()
            pltpu.make_async_copy(scale_ref.at[pl.ds(0, 128)],
                                  m_b[par].at[sl], sem_g[par]).wait()

    def compute(h, par):
        def kb_body(kb, c2):
            k0 = kb * L
            m16 = m_b[par][pl.ds(k0, L)]
            for hh in range(EMBED_DIM // L):
                v = [rows_b[par][k0 + i, pl.ds(hh * L, L)] for i in range(L)]
                t = _transpose16(v, stages)
                for jj in range(L):
                    out_b[par][hh * L + jj, pl.ds(k0, L)] = t[jj] * m16
            return c2

        lax.fori_loop(0, C // L, kb_body, 0)

    fire_gathers(0, 0)

    def g_body(g, carry):
        for par in range(2):
            h = 2 * g + par
            nxt = 1 - par
            if par == 0:
                fire_gathers(h + 1, nxt)
            else:
                @pl.when(g < HIST // 2 - 1)
                def _():
                    fire_gathers(h + 1, nxt)
            wait_gathers(par)

            @pl.when(g >= 1)
            def _():
                pltpu.make_async_copy(
                    out_b[par], out_ref.at[0, :, pl.ds(b0, C)], sem_o[par]
                ).wait()

            compute(h, par)
            pltpu.async_copy(out_b[par], out_ref.at[h, :, pl.ds(b0, C)],
                             sem_o[par])
        return carry

    lax.fori_loop(0, HIST // 2, g_body, 0)
    for par in range(2):
        pltpu.make_async_copy(out_b[par], out_ref.at[0, :, pl.ds(b0, C)],
                              sem_o[par]).wait()


@functools.partial(
    pl.kernel,
    out_type=jax.ShapeDtypeStruct((HIST, EMBED_DIM, BATCH), jnp.float32),
    mesh=plsc.VectorSubcoreMesh(core_axis_name="c", subcore_axis_name="s"),
    scratch_types=[
        pltpu.VMEM((C,), jnp.int32),
        pltpu.VMEM((C,), jnp.int32),
        pltpu.VMEM((C, EMBED_DIM), jnp.float32),
        pltpu.VMEM((C, EMBED_DIM), jnp.float32),
        pltpu.VMEM((C,), jnp.float32),
        pltpu.VMEM((C,), jnp.float32),
        pltpu.VMEM((EMBED_DIM, C), jnp.float32),
        pltpu.VMEM((EMBED_DIM, C), jnp.float32),
        pltpu.SemaphoreType.DMA,
        pltpu.SemaphoreType.DMA,
        pltpu.SemaphoreType.DMA,
        pltpu.SemaphoreType.DMA,
    ],
    compiler_params=pltpu.CompilerParams(use_tc_tiling_on_sc=False),
)
def _sc_lookup(words_ref, weight_ref, scale_ref, out_ref,
               idx0, idx1, rows0, rows1, m0, m1, outv0, outv1,
               sg0, sg1, so0, so1):
    _body(words_ref, weight_ref, scale_ref, out_ref,
          (idx0, idx1), (rows0, rows1), (m0, m1), (outv0, outv1),
          (sg0, sg1), (so0, so1))


SB = 256
NSB_FULL = VOCAB // SB
NSB_MAIN = (NSB_FULL // NW) & ~1


def _tr_body(wt_ref, tail_ref, out_ref, in_b, out_b, sem_i, sem_o):
    wid = lax.axis_index("s") * NC + lax.axis_index("c")
    stages = _make_stages()

    def sb_col0(i):
        return (wid + NW * i) * SB

    def fire_in(i, par):
        pltpu.async_copy(wt_ref.at[:, pl.ds(sb_col0(i), SB)], in_b[par],
                         sem_i[par])

    def wait_in(par):
        pltpu.make_async_copy(wt_ref.at[:, pl.ds(0, SB)], in_b[par],
                              sem_i[par]).wait()

    def compute_sb(par):
        def sub_body(sub, c2):
            for hh in range(EMBED_DIM // L):
                v = [in_b[par][hh * L + i, pl.ds(sub * L, L)]
                     for i in range(L)]
                t = _transpose16(v, stages)
                for jj in range(L):
                    out_b[par][pl.ds((sub * L + jj) * EMBED_DIM + hh * L, L)] \
                        = t[jj]
            return c2

        lax.fori_loop(0, SB // L, sub_body, 0)

    fire_in(0, 0)

    def g_body(g, carry):
        for par in range(2):
            i = 2 * g + par
            nxt = 1 - par
            if par == 0:
                fire_in(i + 1, nxt)
            else:
                @pl.when(g < NSB_MAIN // 2 - 1)
                def _():
                    fire_in(i + 1, nxt)
            wait_in(par)

            @pl.when(g >= 1)
            def _():
                pltpu.make_async_copy(out_b[par],
                                      out_ref.at[pl.ds(0, SB * EMBED_DIM)],
                                      sem_o[par]).wait()

            compute_sb(par)
            pltpu.async_copy(out_b[par],
                             out_ref.at[pl.ds(sb_col0(i) * EMBED_DIM,
                                              SB * EMBED_DIM)],
                             sem_o[par])
        return carry

    lax.fori_loop(0, NSB_MAIN // 2, g_body, 0)
    for par in range(2):
        pltpu.make_async_copy(out_b[par], out_ref.at[pl.ds(0, SB * EMBED_DIM)],
                              sem_o[par]).wait()

    @pl.when(wid < NSB_FULL - NW * NSB_MAIN)
    def _():
        pltpu.sync_copy(wt_ref.at[:, pl.ds(sb_col0(NSB_MAIN), SB)], in_b[0])
        compute_sb(0)
        pltpu.sync_copy(out_b[0],
                        out_ref.at[pl.ds(sb_col0(NSB_MAIN) * EMBED_DIM,
                                         SB * EMBED_DIM)])

    @pl.when(wid == 0)
    def _():
        pltpu.sync_copy(tail_ref,
                        out_ref.at[pl.ds(NSB_FULL * SB * EMBED_DIM,
                                         (VOCAB - NSB_FULL * SB) * EMBED_DIM)])


@functools.partial(
    pl.kernel,
    out_type=jax.ShapeDtypeStruct((VOCAB * EMBED_DIM,), jnp.float32),
    mesh=plsc.VectorSubcoreMesh(core_axis_name="c", subcore_axis_name="s"),
    scratch_types=[
        pltpu.VMEM((EMBED_DIM, SB), jnp.float32),
        pltpu.VMEM((EMBED_DIM, SB), jnp.float32),
        pltpu.VMEM((SB * EMBED_DIM,), jnp.float32),
        pltpu.VMEM((SB * EMBED_DIM,), jnp.float32),
        pltpu.SemaphoreType.DMA,
        pltpu.SemaphoreType.DMA,
        pltpu.SemaphoreType.DMA,
        pltpu.SemaphoreType.DMA,
    ],
    compiler_params=pltpu.CompilerParams(use_tc_tiling_on_sc=True),
)
def _sc_table_rowmajor(wt_ref, tail_ref, out_ref, in0, in1, ob0, ob1, si0,
                       si1, so0, so1):
    _tr_body(wt_ref, tail_ref, out_ref, (in0, in1), (ob0, ob1), (si0, si1),
             (so0, so1))


def kernel(words, weight):
    mask_key = jax.random.fold_in(jax.random.key(0), 12345)
    keep = jax.random.bernoulli(mask_key, p=1.0 - DROPOUT, shape=(VOCAB, 1))
    scale = keep[:, 0].astype(jnp.float32) * (1.0 / (1.0 - DROPOUT))
    words_t = words.T.astype(jnp.int32)
    w_tail = weight[NSB_FULL * SB:, :].reshape(-1)
    w_lin = _sc_table_rowmajor(weight.T, w_tail)
    w_rm = w_lin.reshape(VOCAB, EMBED_DIM)
    out_t = _sc_lookup(words_t, w_rm, scale)
    return jnp.transpose(out_t, (2, 0, 1))


# speedup vs baseline: 1.1308x; 1.0064x over previous
import functools

import jax
import jax.numpy as jnp
from jax import lax
from jax.experimental import pallas as pl
from jax.experimental.pallas import tpu as pltpu
from jax.experimental.pallas import tpu_sc as plsc

VOCAB = 1000000
EMBED_DIM = 32
BATCH = 16384
HIST = 50
DROPOUT = 0.1

NC = 2
NS = 16
NW = NC * NS
L = 16

C = BATCH // NW
J = C // 128


def _make_stages():
    lane = lax.iota(jnp.int32, L)
    stages = []
    for s in (1, 2, 4, 8):
        stages.append((
            s,
            (lane & s) == 0,
            (lane - s) & (L - 1),
            (lane + s) & (L - 1),
        ))
    return stages


def _transpose16(v, stages):
    for s, keep, idxm, idxp in stages:
        nv = list(v)
        for i in range(L):
            if i & s:
                continue
            a, b = v[i], v[i | s]
            bg = b.at[idxm].get(mode="promise_in_bounds")
            ag = a.at[idxp].get(mode="promise_in_bounds")
            nv[i] = jnp.where(keep, a, bg)
            nv[i | s] = jnp.where(keep, ag, b)
        v = nv
    return v


def _body(words_ref, weight_ref, scale_ref, out_ref,
          idx_b, rows_b, m_b, out_b, sem_g, sem_o):
    wid = lax.axis_index("s") * NC + lax.axis_index("c")
    b0 = wid * C
    stages = _make_stages()

    def fire_gathers(h, par):
        pltpu.sync_copy(words_ref.at[h, pl.ds(b0, C)], idx_b[par])
        for j in range(J):
            sl = pl.ds(j * 128, 128)
            pltpu.async_copy(weight_ref.at[idx_b[par].at[sl]],
                             rows_b[par].at[sl], sem_g[par])
            pltpu.async_copy(scale_ref.at[idx_b[par].at[sl]],
                             m_b[par].at[sl], sem_g[par])

    def wait_gathers(par):
        for j in range(J):
            sl = pl.ds(j * 128, 128)
            pltpu.make_async_copy(weight_ref.at[pl.ds(0, 128)],
                                  rows_b[par].at[sl], sem_g[par]).wait()
            pltpu.make_async_copy(scale_ref.at[pl.ds(0, 128)],
                                  m_b[par].at[sl], sem_g[par]).wait()

    def compute(h, par):
        def kb_body(kb, c2):
            k0 = kb * L
            m16 = m_b[par][pl.ds(k0, L)]
            for hh in range(EMBED_DIM // L):
                v = [rows_b[par][k0 + i, pl.ds(hh * L, L)] for i in range(L)]
                t = _transpose16(v, stages)
                for jj in range(L):
                    out_b[par][hh * L + jj, pl.ds(k0, L)] = t[jj] * m16
            return c2

        lax.fori_loop(0, C // L, kb_body, 0)

    fire_gathers(0, 0)

    def g_body(g, carry):
        for par in range(2):
            h = 2 * g + par
            nxt = 1 - par
            if par == 0:
                fire_gathers(h + 1, nxt)
            else:
                @pl.when(g < HIST // 2 - 1)
                def _():
                    fire_gathers(h + 1, nxt)
            wait_gathers(par)

            @pl.when(g >= 1)
            def _():
                pltpu.make_async_copy(
                    out_b[par], out_ref.at[0, :, pl.ds(b0, C)], sem_o[par]
                ).wait()

            compute(h, par)
            pltpu.async_copy(out_b[par], out_ref.at[h, :, pl.ds(b0, C)],
                             sem_o[par])
        return carry

    lax.fori_loop(0, HIST // 2, g_body, 0)
    for par in range(2):
        pltpu.make_async_copy(out_b[par], out_ref.at[0, :, pl.ds(b0, C)],
                              sem_o[par]).wait()


@functools.partial(
    pl.kernel,
    out_type=jax.ShapeDtypeStruct((HIST, EMBED_DIM, BATCH), jnp.float32),
    mesh=plsc.VectorSubcoreMesh(core_axis_name="c", subcore_axis_name="s"),
    scratch_types=[
        pltpu.VMEM((C,), jnp.int32),
        pltpu.VMEM((C,), jnp.int32),
        pltpu.VMEM((C, EMBED_DIM), jnp.float32),
        pltpu.VMEM((C, EMBED_DIM), jnp.float32),
        pltpu.VMEM((C,), jnp.float32),
        pltpu.VMEM((C,), jnp.float32),
        pltpu.VMEM((EMBED_DIM, C), jnp.float32),
        pltpu.VMEM((EMBED_DIM, C), jnp.float32),
        pltpu.SemaphoreType.DMA,
        pltpu.SemaphoreType.DMA,
        pltpu.SemaphoreType.DMA,
        pltpu.SemaphoreType.DMA,
    ],
    compiler_params=pltpu.CompilerParams(use_tc_tiling_on_sc=False),
)
def _sc_lookup(words_ref, weight_ref, scale_ref, out_ref,
               idx0, idx1, rows0, rows1, m0, m1, outv0, outv1,
               sg0, sg1, so0, so1):
    _body(words_ref, weight_ref, scale_ref, out_ref,
          (idx0, idx1), (rows0, rows1), (m0, m1), (outv0, outv1),
          (sg0, sg1), (so0, so1))


SB = 512
NSB_FULL = VOCAB // SB
NSB_MAIN = (NSB_FULL // NW) & ~1


def _tr_body(wt_ref, tail_ref, out_ref, in_b, out_b, sem_i, sem_o):
    wid = lax.axis_index("s") * NC + lax.axis_index("c")
    stages = _make_stages()

    def sb_col0(i):
        return (wid + NW * i) * SB

    def fire_in(i, par):
        pltpu.async_copy(wt_ref.at[:, pl.ds(sb_col0(i), SB)], in_b[par],
                         sem_i[par])

    def wait_in(par):
        pltpu.make_async_copy(wt_ref.at[:, pl.ds(0, SB)], in_b[par],
                              sem_i[par]).wait()

    def compute_sb(par):
        def sub_body(sub, c2):
            for hh in range(EMBED_DIM // L):
                v = [in_b[par][hh * L + i, pl.ds(sub * L, L)]
                     for i in range(L)]
                t = _transpose16(v, stages)
                for jj in range(L):
                    out_b[par][pl.ds((sub * L + jj) * EMBED_DIM + hh * L, L)] \
                        = t[jj]
            return c2

        lax.fori_loop(0, SB // L, sub_body, 0)

    fire_in(0, 0)

    def g_body(g, carry):
        for par in range(2):
            i = 2 * g + par
            nxt = 1 - par
            if par == 0:
                fire_in(i + 1, nxt)
            else:
                @pl.when(g < NSB_MAIN // 2 - 1)
                def _():
                    fire_in(i + 1, nxt)
            wait_in(par)

            @pl.when(g >= 1)
            def _():
                pltpu.make_async_copy(out_b[par],
                                      out_ref.at[pl.ds(0, SB * EMBED_DIM)],
                                      sem_o[par]).wait()

            compute_sb(par)
            pltpu.async_copy(out_b[par],
                             out_ref.at[pl.ds(sb_col0(i) * EMBED_DIM,
                                              SB * EMBED_DIM)],
                             sem_o[par])
        return carry

    lax.fori_loop(0, NSB_MAIN // 2, g_body, 0)
    for par in range(2):
        pltpu.make_async_copy(out_b[par], out_ref.at[pl.ds(0, SB * EMBED_DIM)],
                              sem_o[par]).wait()

    for r in range(-(-(NSB_FULL - NW * NSB_MAIN) // NW)):
        @pl.when(wid + NW * (NSB_MAIN + r) < NSB_FULL)
        def _():
            pltpu.sync_copy(wt_ref.at[:, pl.ds(sb_col0(NSB_MAIN + r), SB)],
                            in_b[0])
            compute_sb(0)
            pltpu.sync_copy(out_b[0],
                            out_ref.at[pl.ds(sb_col0(NSB_MAIN + r) * EMBED_DIM,
                                             SB * EMBED_DIM)])

    @pl.when(wid == 0)
    def _():
        pltpu.sync_copy(tail_ref,
                        out_ref.at[pl.ds(NSB_FULL * SB * EMBED_DIM,
                                         (VOCAB - NSB_FULL * SB) * EMBED_DIM)])


@functools.partial(
    pl.kernel,
    out_type=jax.ShapeDtypeStruct((VOCAB * EMBED_DIM,), jnp.float32),
    mesh=plsc.VectorSubcoreMesh(core_axis_name="c", subcore_axis_name="s"),
    scratch_types=[
        pltpu.VMEM((EMBED_DIM, SB), jnp.float32),
        pltpu.VMEM((EMBED_DIM, SB), jnp.float32),
        pltpu.VMEM((SB * EMBED_DIM,), jnp.float32),
        pltpu.VMEM((SB * EMBED_DIM,), jnp.float32),
        pltpu.SemaphoreType.DMA,
        pltpu.SemaphoreType.DMA,
        pltpu.SemaphoreType.DMA,
        pltpu.SemaphoreType.DMA,
    ],
    compiler_params=pltpu.CompilerParams(use_tc_tiling_on_sc=True),
)
def _sc_table_rowmajor(wt_ref, tail_ref, out_ref, in0, in1, ob0, ob1, si0,
                       si1, so0, so1):
    _tr_body(wt_ref, tail_ref, out_ref, (in0, in1), (ob0, ob1), (si0, si1),
             (so0, so1))


def kernel(words, weight):
    mask_key = jax.random.fold_in(jax.random.key(0), 12345)
    keep = jax.random.bernoulli(mask_key, p=1.0 - DROPOUT, shape=(VOCAB, 1))
    scale = keep[:, 0].astype(jnp.float32) * (1.0 / (1.0 - DROPOUT))
    words_t = words.T.astype(jnp.int32)
    w_tail = weight[NSB_FULL * SB:, :].reshape(-1)
    w_lin = _sc_table_rowmajor(weight.T, w_tail)
    w_rm = w_lin.reshape(VOCAB, EMBED_DIM)
    out_t = _sc_lookup(words_t, w_rm, scale)
    return jnp.transpose(out_t, (2, 0, 1))


# speedup vs baseline: 1.4877x; 1.3156x over previous
import functools

import jax
import jax.numpy as jnp
from jax import lax
from jax.experimental import pallas as pl
from jax.experimental.pallas import tpu as pltpu
from jax.experimental.pallas import tpu_sc as plsc

VOCAB = 1000000
EMBED_DIM = 32
BATCH = 16384
HIST = 50
DROPOUT = 0.1

NC = 2
NS = 16
NW = NC * NS
L = 16

C = BATCH // NW
J = C // 128


def _make_stages():
    lane = lax.iota(jnp.int32, L)
    stages = []
    for s in (1, 2, 4, 8):
        stages.append((
            s,
            (lane & s) == 0,
            (lane - s) & (L - 1),
            (lane + s) & (L - 1),
        ))
    return stages


def _transpose16(v, stages):
    for s, keep, idxm, idxp in stages:
        nv = list(v)
        for i in range(L):
            if i & s:
                continue
            a, b = v[i], v[i | s]
            bg = b.at[idxm].get(mode="promise_in_bounds")
            ag = a.at[idxp].get(mode="promise_in_bounds")
            nv[i] = jnp.where(keep, a, bg)
            nv[i | s] = jnp.where(keep, ag, b)
        v = nv
    return v


def _body(words_ref, weight_ref, scale_ref, out_ref,
          idx_b, rows_b, m_b, out_b, sem_g, sem_o):
    wid = lax.axis_index("s") * NC + lax.axis_index("c")
    b0 = wid * C
    stages = _make_stages()

    def fire_gathers(h, par):
        pltpu.sync_copy(words_ref.at[h, pl.ds(b0, C)], idx_b[par])
        for j in range(J):
            sl = pl.ds(j * 128, 128)
            pltpu.async_copy(weight_ref.at[idx_b[par].at[sl]],
                             rows_b[par].at[sl], sem_g[par])
            pltpu.async_copy(scale_ref.at[idx_b[par].at[sl]],
                             m_b[par].at[sl], sem_g[par])

    def wait_gathers(par):
        for j in range(J):
            sl = pl.ds(j * 128, 128)
            pltpu.make_async_copy(weight_ref.at[pl.ds(0, 128)],
                                  rows_b[par].at[sl], sem_g[par]).wait()
            pltpu.make_async_copy(scale_ref.at[pl.ds(0, 128)],
                                  m_b[par].at[sl], sem_g[par]).wait()

    def compute(h, par):
        def kb_body(kb, c2):
            k0 = kb * L
            tc = kb // 8
            l0 = (kb % 8) * L
            m16 = m_b[par][pl.ds(k0, L)]
            for hh in range(EMBED_DIM // L):
                v = [rows_b[par][k0 + i, pl.ds(hh * L, L)] for i in range(L)]
                t = _transpose16(v, stages)
                for jj in range(L):
                    out_b[par][hh * 2 + jj // 8, tc, jj % 8, pl.ds(l0, L)] \
                        = t[jj] * m16
            return c2

        lax.fori_loop(0, C // L, kb_body, 0)

    fire_gathers(0, 0)

    def g_body(g, carry):
        for par in range(2):
            h = 2 * g + par
            nxt = 1 - par
            if par == 0:
                fire_gathers(h + 1, nxt)
            else:
                @pl.when(g < HIST // 2 - 1)
                def _():
                    fire_gathers(h + 1, nxt)
            wait_gathers(par)

            @pl.when(g >= 1)
            def _():
                for tr in range(EMBED_DIM // 8):
                    pltpu.make_async_copy(
                        out_b[par].at[tr],
                        out_ref.at[0, tr, pl.ds(0, C // 128)], sem_o[par]
                    ).wait()

            compute(h, par)
            for tr in range(EMBED_DIM // 8):
                pltpu.async_copy(out_b[par].at[tr],
                                 out_ref.at[h, tr, pl.ds(wid * (C // 128),
                                                         C // 128)],
                                 sem_o[par])
        return carry

    lax.fori_loop(0, HIST // 2, g_body, 0)
    for par in range(2):
        for tr in range(EMBED_DIM // 8):
            pltpu.make_async_copy(out_b[par].at[tr],
                                  out_ref.at[0, tr, pl.ds(0, C // 128)],
                                  sem_o[par]).wait()


@functools.partial(
    pl.kernel,
    out_type=jax.ShapeDtypeStruct(
        (HIST, EMBED_DIM // 8, BATCH // 128, 8, 128), jnp.float32),
    mesh=plsc.VectorSubcoreMesh(core_axis_name="c", subcore_axis_name="s"),
    scratch_types=[
        pltpu.VMEM((C,), jnp.int32),
        pltpu.VMEM((C,), jnp.int32),
        pltpu.VMEM((C, EMBED_DIM), jnp.float32),
        pltpu.VMEM((C, EMBED_DIM), jnp.float32),
        pltpu.VMEM((C,), jnp.float32),
        pltpu.VMEM((C,), jnp.float32),
        pltpu.VMEM((EMBED_DIM // 8, C // 128, 8, 128), jnp.float32),
        pltpu.VMEM((EMBED_DIM // 8, C // 128, 8, 128), jnp.float32),
        pltpu.SemaphoreType.DMA,
        pltpu.SemaphoreType.DMA,
        pltpu.SemaphoreType.DMA,
        pltpu.SemaphoreType.DMA,
    ],
    compiler_params=pltpu.CompilerParams(use_tc_tiling_on_sc=False),
)
def _sc_lookup(words_ref, weight_ref, scale_ref, out_ref,
               idx0, idx1, rows0, rows1, m0, m1, outv0, outv1,
               sg0, sg1, so0, so1):
    _body(words_ref, weight_ref, scale_ref, out_ref,
          (idx0, idx1), (rows0, rows1), (m0, m1), (outv0, outv1),
          (sg0, sg1), (so0, so1))


SB = 512
NSB_FULL = VOCAB // SB
NSB_MAIN = (NSB_FULL // NW) & ~1


def _tr_body(wt_ref, tail_ref, out_ref, in_b, out_b, sem_i, sem_o):
    wid = lax.axis_index("s") * NC + lax.axis_index("c")
    stages = _make_stages()

    def sb_col0(i):
        return (wid + NW * i) * SB

    def fire_in(i, par):
        pltpu.async_copy(wt_ref.at[:, pl.ds(sb_col0(i), SB)], in_b[par],
                         sem_i[par])

    def wait_in(par):
        pltpu.make_async_copy(wt_ref.at[:, pl.ds(0, SB)], in_b[par],
                              sem_i[par]).wait()

    def compute_sb(par):
        def sub_body(sub, c2):
            for hh in range(EMBED_DIM // L):
                v = [in_b[par][hh * L + i, pl.ds(sub * L, L)]
                     for i in range(L)]
                t = _transpose16(v, stages)
                for jj in range(L):
                    out_b[par][pl.ds((sub * L + jj) * EMBED_DIM + hh * L, L)] \
                        = t[jj]
            return c2

        lax.fori_loop(0, SB // L, sub_body, 0)

    fire_in(0, 0)

    def g_body(g, carry):
        for par in range(2):
            i = 2 * g + par
            nxt = 1 - par
            if par == 0:
                fire_in(i + 1, nxt)
            else:
                @pl.when(g < NSB_MAIN // 2 - 1)
                def _():
                    fire_in(i + 1, nxt)
            wait_in(par)

            @pl.when(g >= 1)
            def _():
                pltpu.make_async_copy(out_b[par],
                                      out_ref.at[pl.ds(0, SB * EMBED_DIM)],
                                      sem_o[par]).wait()

            compute_sb(par)
            pltpu.async_copy(out_b[par],
                             out_ref.at[pl.ds(sb_col0(i) * EMBED_DIM,
                                              SB * EMBED_DIM)],
                             sem_o[par])
        return carry

    lax.fori_loop(0, NSB_MAIN // 2, g_body, 0)
    for par in range(2):
        pltpu.make_async_copy(out_b[par], out_ref.at[pl.ds(0, SB * EMBED_DIM)],
                              sem_o[par]).wait()

    for r in range(-(-(NSB_FULL - NW * NSB_MAIN) // NW)):
        @pl.when(wid + NW * (NSB_MAIN + r) < NSB_FULL)
        def _():
            pltpu.sync_copy(wt_ref.at[:, pl.ds(sb_col0(NSB_MAIN + r), SB)],
                            in_b[0])
            compute_sb(0)
            pltpu.sync_copy(out_b[0],
                            out_ref.at[pl.ds(sb_col0(NSB_MAIN + r) * EMBED_DIM,
                                             SB * EMBED_DIM)])

    @pl.when(wid == 0)
    def _():
        pltpu.sync_copy(tail_ref,
                        out_ref.at[pl.ds(NSB_FULL * SB * EMBED_DIM,
                                         (VOCAB - NSB_FULL * SB) * EMBED_DIM)])


@functools.partial(
    pl.kernel,
    out_type=jax.ShapeDtypeStruct((VOCAB * EMBED_DIM,), jnp.float32),
    mesh=plsc.VectorSubcoreMesh(core_axis_name="c", subcore_axis_name="s"),
    scratch_types=[
        pltpu.VMEM((EMBED_DIM, SB), jnp.float32),
        pltpu.VMEM((EMBED_DIM, SB), jnp.float32),
        pltpu.VMEM((SB * EMBED_DIM,), jnp.float32),
        pltpu.VMEM((SB * EMBED_DIM,), jnp.float32),
        pltpu.SemaphoreType.DMA,
        pltpu.SemaphoreType.DMA,
        pltpu.SemaphoreType.DMA,
        pltpu.SemaphoreType.DMA,
    ],
    compiler_params=pltpu.CompilerParams(use_tc_tiling_on_sc=True),
)
def _sc_table_rowmajor(wt_ref, tail_ref, out_ref, in0, in1, ob0, ob1, si0,
                       si1, so0, so1):
    _tr_body(wt_ref, tail_ref, out_ref, (in0, in1), (ob0, ob1), (si0, si1),
             (so0, so1))


def kernel(words, weight):
    mask_key = jax.random.fold_in(jax.random.key(0), 12345)
    keep = jax.random.bernoulli(mask_key, p=1.0 - DROPOUT, shape=(VOCAB, 1))
    scale = keep[:, 0].astype(jnp.float32) * (1.0 / (1.0 - DROPOUT))
    words_t = words.T.astype(jnp.int32)
    w_tail = weight[NSB_FULL * SB:, :].reshape(-1)
    w_lin = _sc_table_rowmajor(weight.T, w_tail)
    w_rm = w_lin.reshape(VOCAB, EMBED_DIM)
    x5 = _sc_lookup(words_t, w_rm, scale)
    out = jnp.transpose(x5, (2, 4, 0, 1, 3)).reshape(BATCH, HIST, EMBED_DIM)
    return out
